# baseline (speedup 1.0000x reference)
import functools

import jax
import jax.numpy as jnp
from jax import lax
from jax.experimental import pallas as pl
from jax.experimental.pallas import tpu as pltpu
from jax.experimental.pallas import tpu_sc as plsc

ENTITY_LEN = 1000000
REL_LEN = 1000
EMB_DIM = 50
BATCH = 16384

_NC, _NS = 2, 16
_NW = _NC * _NS
_RPW = BATCH // _NW
_RCHUNK = 128
_NRCHUNK = _RPW // _RCHUNK
_EROWS = 64
_NECHUNK = _RPW // _EROWS
_EVIEW_ROWS = ENTITY_LEN * EMB_DIM // 128
_MAXQ = _EVIEW_ROWS - 1


def _sc_gather6(e128, relpad, h, t, h2, t2, r2d, r2_2d):
    mesh = plsc.VectorSubcoreMesh(core_axis_name="c", subcore_axis_name="s")
    ent_out = jax.ShapeDtypeStruct((BATCH, EMB_DIM), jnp.float32)
    rel_out = jax.ShapeDtypeStruct((BATCH, 128), jnp.float32)

    @functools.partial(
        pl.kernel,
        mesh=mesh,
        out_type=[ent_out] * 4 + [rel_out] * 2,
        scratch_types=[
            pltpu.VMEM((_RPW,), jnp.int32),
            pltpu.VMEM((_RPW,), jnp.int32),
            pltpu.VMEM((_NECHUNK, 2 * _EROWS), jnp.int32),
            pltpu.VMEM((2 * _EROWS, 128), jnp.float32),
            pltpu.VMEM((_EROWS, EMB_DIM), jnp.float32),
            pltpu.VMEM((_NRCHUNK, _RCHUNK), jnp.int32),
            pltpu.VMEM((_RCHUNK, 128), jnp.float32),
            pltpu.SemaphoreType.DMA,
            pltpu.SemaphoreType.DMA,
        ],
    )
    def k(e_hbm, rel_hbm, h_h, t_h, h2_h, t2_h, r_h, r2_h,
          o_he, o_te, o_he2, o_te2, o_re, o_re2,
          idx_v, off_v, q_v, pairs_v, out_v, ridx_v, rrows_v, sem, sem2):
        wid = lax.axis_index("s") * _NC + lax.axis_index("c")
        base = wid * _RPW
        iota = lax.iota(jnp.int32, 16)

        for idx_hbm, out_hbm in ((h_h, o_he), (t_h, o_te),
                                 (h2_h, o_he2), (t2_h, o_te2)):
            pltpu.sync_copy(idx_hbm.at[pl.ds(base, _RPW)], idx_v)

            for g in range(_RPW // 16):
                v = idx_v[pl.ds(g * 16, 16)]
                f = v * 50
                q0 = lax.shift_right_logical(f, 7)
                q1 = jnp.minimum(q0 + 1, _MAXQ)
                off_v[pl.ds(g * 16, 16)] = jnp.bitwise_and(f, 127)
                cc = g // 4
                p = (g % 4) * 16
                q_v[cc, pl.ds(p, 16)] = q0
                q_v[cc, pl.ds(_EROWS + p, 16)] = q1

            def chunk(c, _):
                pltpu.async_copy(e_hbm.at[q_v.at[c]], pairs_v, sem).wait()
                for g in range(_EROWS // 16):
                    offs = off_v[pl.ds(c * _EROWS + g * 16, 16)]
                    for j in range(16):
                        r = g * 16 + j
                        o = offs[j]
                        po = o + iota
                        for seg in (0, 16, 32, 34):
                            pos = po + seg
                            m = pos < 128
                            a = pairs_v[r, pl.ds(o + seg, 16)]
                            b = pairs_v[_EROWS + r, pl.ds(o + seg - 128, 16)]
                            out_v[r, pl.ds(seg, 16)] = jnp.where(m, a, b)
                pltpu.sync_copy(out_v, out_hbm.at[pl.ds(base + c * _EROWS, _EROWS)])
                return 0

            lax.fori_loop(0, _NECHUNK, chunk, 0)

        for idx_hbm, out_hbm in ((r_h, o_re), (r2_h, o_re2)):
            pltpu.sync_copy(idx_hbm.at[pl.ds(wid * _NRCHUNK, _NRCHUNK)], ridx_v)
            for j in range(_NRCHUNK):
                pltpu.async_copy(rel_hbm.at[ridx_v.at[j]], rrows_v, sem2).wait()
                pltpu.sync_copy(
                    rrows_v, out_hbm.at[pl.ds(base + j * _RCHUNK, _RCHUNK)])

    return k(e128, relpad, h, t, h2, t2, r2d, r2_2d)


_TC_ROWS = 512
_TC_BLOCKS = BATCH // _TC_ROWS


def _tc_body(he, te, he2, te2, re, re2, og, ob, acc):
    p = pl.program_id(0)
    i = pl.program_id(1)
    ins = (he, te, he2, te2, re, re2)

    @pl.when((p == 0) & (i == 0))
    def _init():
        acc[...] = jnp.zeros_like(acc)

    @pl.when(p == 0)
    def _stats():
        for k, x in enumerate(ins):
            xv = x[...][:, :EMB_DIM]
            acc[2 * k, :] += jnp.sum(xv, axis=0)
            acc[2 * k + 1, :] += jnp.sum(xv * xv, axis=0)

    @pl.when(p == 1)
    def _score():
        inv_b = 1.0 / BATCH
        norm = []
        for k, x in enumerate(ins):
            m = (acc[2 * k, :] * inv_b)[None, :]
            v = (acc[2 * k + 1, :] * inv_b)[None, :] - m * m
            norm.append((x[...][:, :EMB_DIM] - m) * lax.rsqrt(v + 1e-5))
        og[...] = jnp.sum(jnp.abs(norm[0] + norm[4] - norm[1]), axis=1, keepdims=True)
        ob[...] = jnp.sum(jnp.abs(norm[2] + norm[5] - norm[3]), axis=1, keepdims=True)


def _tc_bn_score(he, te, he2, te2, re, re2):
    ent_spec = pl.BlockSpec((_TC_ROWS, EMB_DIM), lambda p, i: (i, 0))
    rel_spec = pl.BlockSpec((_TC_ROWS, 128), lambda p, i: (i, 0))
    out_spec = pl.BlockSpec((_TC_ROWS, 1), lambda p, i: (i, 0))
    return pl.pallas_call(
        _tc_body,
        grid=(2, _TC_BLOCKS),
        in_specs=[ent_spec] * 4 + [rel_spec] * 2,
        out_specs=[out_spec] * 2,
        out_shape=[jax.ShapeDtypeStruct((BATCH, 1), jnp.float32)] * 2,
        scratch_shapes=[pltpu.VMEM((12, EMB_DIM), jnp.float32)],
    )(he, te, he2, te2, re, re2)


def kernel(h, t, r, h_2, t_2, r_2, e_emb, rel_emb):
    e128 = e_emb.reshape(_EVIEW_ROWS, 128)
    relpad = jnp.pad(rel_emb, ((0, 0), (0, 128 - EMB_DIM)))
    i32 = lambda x: x.astype(jnp.int32)
    r2d = i32(r).reshape(_NW * _NRCHUNK, _RCHUNK)
    r2_2d = i32(r_2).reshape(_NW * _NRCHUNK, _RCHUNK)
    he, te, he2, te2, re, re2 = _sc_gather6(
        e128, relpad, i32(h), i32(t), i32(h_2), i32(t_2), r2d, r2_2d)
    sg, sb = _tc_bn_score(he, te, he2, te2, re, re2)
    return (sg.reshape(BATCH), sb.reshape(BATCH))


# speedup vs baseline: 3.4076x; 3.4076x over previous
import functools

import jax
import jax.numpy as jnp
from jax import lax
from jax.experimental import pallas as pl
from jax.experimental.pallas import tpu as pltpu
from jax.experimental.pallas import tpu_sc as plsc

ENTITY_LEN = 1000000
REL_LEN = 1000
EMB_DIM = 50
BATCH = 16384

_NC, _NS = 2, 16
_NW = _NC * _NS
_RPW = BATCH // _NW
_CHUNK = 128
_NCHUNK = _RPW // _CHUNK
_SLAB = 256
_NSLAB = _RPW // _SLAB


def _sc_gather6(e_emb, relpad, h, t, h2, t2, r2d, r2_2d):
    mesh = plsc.VectorSubcoreMesh(core_axis_name="c", subcore_axis_name="s")
    ent_out = jax.ShapeDtypeStruct((BATCH, EMB_DIM), jnp.float32)
    rel_out = jax.ShapeDtypeStruct((BATCH, 128), jnp.float32)

    @functools.partial(
        pl.kernel,
        mesh=mesh,
        out_type=[ent_out] * 4 + [rel_out] * 2,
        scratch_types=[
            pltpu.VMEM((4, _RPW), jnp.int32),
            pltpu.VMEM((_SLAB, EMB_DIM), jnp.float32),
            pltpu.VMEM((_SLAB, EMB_DIM), jnp.float32),
            pltpu.VMEM((_NCHUNK, _CHUNK), jnp.int32),
            pltpu.VMEM((_CHUNK, 128), jnp.float32),
            pltpu.SemaphoreType.DMA,
            pltpu.SemaphoreType.DMA,
            pltpu.SemaphoreType.DMA,
        ],
    )
    def k(e_hbm, rel_hbm, h_h, t_h, h2_h, t2_h, r_h, r2_h,
          o_he, o_te, o_he2, o_te2, o_re, o_re2,
          idx4, rows_a, rows_b, idx_v, rrows_v, sem_a, sem_b, sem_r):
        wid = lax.axis_index("s") * _NC + lax.axis_index("c")
        base = wid * _RPW
        ent = ((h_h, o_he), (t_h, o_te), (h2_h, o_he2), (t2_h, o_te2))
        for ki, (idx_hbm, _) in enumerate(ent):
            pltpu.sync_copy(idx_hbm.at[pl.ds(base, _RPW)], idx4.at[ki])

        slabs = [(ki, sl, out_hbm)
                 for ki, (_, out_hbm) in enumerate(ent)
                 for sl in range(_NSLAB)]
        bufs = (rows_a, rows_b)
        sems = (sem_a, sem_b)

        def enqueue(s):
            ki, sl, _ = slabs[s]
            buf, sm = bufs[s % 2], sems[s % 2]

            def body(g, _):
                vec = idx4[ki, pl.ds(sl * _SLAB + g * 16, 16)]
                for j in range(16):
                    pltpu.async_copy(
                        e_hbm.at[pl.ds(vec[j], 1)],
                        buf.at[pl.ds(g * 16 + j, 1)], sm)
                return 0

            lax.fori_loop(0, _SLAB // 16, body, 0)

        def drain_wb(s):
            ki, sl, out_hbm = slabs[s]
            buf, sm = bufs[s % 2], sems[s % 2]
            pltpu.make_async_copy(e_hbm.at[pl.ds(0, _SLAB)], buf, sm).wait()
            pltpu.sync_copy(buf, out_hbm.at[pl.ds(base + sl * _SLAB, _SLAB)])

        enqueue(0)
        for s in range(len(slabs) - 1):
            enqueue(s + 1)
            drain_wb(s)
        drain_wb(len(slabs) - 1)

        for idx_hbm, out_hbm in ((r_h, o_re), (r2_h, o_re2)):
            pltpu.sync_copy(idx_hbm.at[pl.ds(wid * _NCHUNK, _NCHUNK)], idx_v)
            for j in range(_NCHUNK):
                pltpu.async_copy(rel_hbm.at[idx_v.at[j]], rrows_v, sem_r).wait()
                pltpu.sync_copy(rrows_v, out_hbm.at[pl.ds(base + j * _CHUNK, _CHUNK)])

    return k(e_emb, relpad, h, t, h2, t2, r2d, r2_2d)


_TC_ROWS = 512
_TC_BLOCKS = BATCH // _TC_ROWS


def _tc_body(he, te, he2, te2, re, re2, og, ob, acc):
    p = pl.program_id(0)
    i = pl.program_id(1)
    ins = (he, te, he2, te2, re, re2)

    @pl.when((p == 0) & (i == 0))
    def _init():
        acc[...] = jnp.zeros_like(acc)

    @pl.when(p == 0)
    def _stats():
        for k, x in enumerate(ins):
            xv = x[...][:, :EMB_DIM]
            acc[2 * k, :] += jnp.sum(xv, axis=0)
            acc[2 * k + 1, :] += jnp.sum(xv * xv, axis=0)

    @pl.when(p == 1)
    def _score():
        inv_b = 1.0 / BATCH
        norm = []
        for k, x in enumerate(ins):
            m = (acc[2 * k, :] * inv_b)[None, :]
            v = (acc[2 * k + 1, :] * inv_b)[None, :] - m * m
            norm.append((x[...][:, :EMB_DIM] - m) * lax.rsqrt(v + 1e-5))
        og[...] = jnp.sum(jnp.abs(norm[0] + norm[4] - norm[1]), axis=1, keepdims=True)
        ob[...] = jnp.sum(jnp.abs(norm[2] + norm[5] - norm[3]), axis=1, keepdims=True)


def _tc_bn_score(he, te, he2, te2, re, re2):
    ent_spec = pl.BlockSpec((_TC_ROWS, EMB_DIM), lambda p, i: (i, 0))
    rel_spec = pl.BlockSpec((_TC_ROWS, 128), lambda p, i: (i, 0))
    out_spec = pl.BlockSpec((_TC_ROWS, 1), lambda p, i: (i, 0))
    return pl.pallas_call(
        _tc_body,
        grid=(2, _TC_BLOCKS),
        in_specs=[ent_spec] * 4 + [rel_spec] * 2,
        out_specs=[out_spec] * 2,
        out_shape=[jax.ShapeDtypeStruct((BATCH, 1), jnp.float32)] * 2,
        scratch_shapes=[pltpu.VMEM((12, EMB_DIM), jnp.float32)],
    )(he, te, he2, te2, re, re2)


def kernel(h, t, r, h_2, t_2, r_2, e_emb, rel_emb):
    relpad = jnp.pad(rel_emb, ((0, 0), (0, 128 - EMB_DIM)))
    i32 = lambda x: x.astype(jnp.int32)
    r2d = i32(r).reshape(_NW * _NCHUNK, _CHUNK)
    r2_2d = i32(r_2).reshape(_NW * _NCHUNK, _CHUNK)
    he, te, he2, te2, re, re2 = _sc_gather6(
        e_emb, relpad, i32(h), i32(t), i32(h_2), i32(t_2), r2d, r2_2d)
    sg, sb = _tc_bn_score(he, te, he2, te2, re, re2)
    return (sg.reshape(BATCH), sb.reshape(BATCH))


# speedup vs baseline: 3.4170x; 1.0027x over previous
import functools

import jax
import jax.numpy as jnp
from jax import lax
from jax.experimental import pallas as pl
from jax.experimental.pallas import tpu as pltpu
from jax.experimental.pallas import tpu_sc as plsc

ENTITY_LEN = 1000000
REL_LEN = 1000
EMB_DIM = 50
BATCH = 16384

_NC, _NS = 2, 16
_NW = _NC * _NS
_RPW = BATCH // _NW
_CHUNK = 128
_NCHUNK = _RPW // _CHUNK
_SLAB = 128
_NSLAB = _RPW // _SLAB
_NBUF = 4


def _sc_gather6(e_emb, relpad, h, t, h2, t2, r2d, r2_2d):
    mesh = plsc.VectorSubcoreMesh(core_axis_name="c", subcore_axis_name="s")
    ent_out = jax.ShapeDtypeStruct((BATCH, EMB_DIM), jnp.float32)
    rel_out = jax.ShapeDtypeStruct((BATCH, 128), jnp.float32)

    @functools.partial(
        pl.kernel,
        mesh=mesh,
        out_type=[ent_out] * 4 + [rel_out] * 2,
        scratch_types=[
            pltpu.VMEM((4, _RPW), jnp.int32),
            pltpu.VMEM((_SLAB, EMB_DIM), jnp.float32),
            pltpu.VMEM((_SLAB, EMB_DIM), jnp.float32),
            pltpu.VMEM((_SLAB, EMB_DIM), jnp.float32),
            pltpu.VMEM((_SLAB, EMB_DIM), jnp.float32),
            pltpu.VMEM((_NCHUNK, _CHUNK), jnp.int32),
            pltpu.VMEM((_CHUNK, 128), jnp.float32),
            pltpu.SemaphoreType.DMA,
            pltpu.SemaphoreType.DMA,
            pltpu.SemaphoreType.DMA,
            pltpu.SemaphoreType.DMA,
            pltpu.SemaphoreType.DMA,
        ],
    )
    def k(e_hbm, rel_hbm, h_h, t_h, h2_h, t2_h, r_h, r2_h,
          o_he, o_te, o_he2, o_te2, o_re, o_re2,
          idx4, rows_a, rows_b, rows_c, rows_d, idx_v, rrows_v,
          sem_a, sem_b, sem_c, sem_d, sem_r):
        wid = lax.axis_index("s") * _NC + lax.axis_index("c")
        base = wid * _RPW
        ent = ((h_h, o_he), (t_h, o_te), (h2_h, o_he2), (t2_h, o_te2))
        for ki, (idx_hbm, _) in enumerate(ent):
            pltpu.sync_copy(idx_hbm.at[pl.ds(base, _RPW)], idx4.at[ki])

        slabs = [(ki, sl, out_hbm)
                 for ki, (_, out_hbm) in enumerate(ent)
                 for sl in range(_NSLAB)]
        bufs = (rows_a, rows_b, rows_c, rows_d)
        sems = (sem_a, sem_b, sem_c, sem_d)

        def enqueue(s):
            ki, sl, _ = slabs[s]
            buf, sm = bufs[s % _NBUF], sems[s % _NBUF]

            def body(g, _):
                vec = idx4[ki, pl.ds(sl * _SLAB + g * 16, 16)]
                for j in range(16):
                    pltpu.async_copy(
                        e_hbm.at[pl.ds(vec[j], 1)],
                        buf.at[pl.ds(g * 16 + j, 1)], sm)
                return 0

            lax.fori_loop(0, _SLAB // 16, body, 0)

        def drain_wb(s):
            ki, sl, out_hbm = slabs[s]
            buf, sm = bufs[s % _NBUF], sems[s % _NBUF]
            pltpu.make_async_copy(e_hbm.at[pl.ds(0, _SLAB)], buf, sm).wait()
            pltpu.sync_copy(buf, out_hbm.at[pl.ds(base + sl * _SLAB, _SLAB)])

        for s in range(_NBUF - 1):
            enqueue(s)

        def rel_gather(a):
            idx_hbm, out_hbm = ((r_h, o_re), (r2_h, o_re2))[a]
            pltpu.sync_copy(idx_hbm.at[pl.ds(wid * _NCHUNK, _NCHUNK)], idx_v)
            for j in range(_NCHUNK):
                pltpu.async_copy(rel_hbm.at[idx_v.at[j]], rrows_v, sem_r).wait()
                pltpu.sync_copy(rrows_v, out_hbm.at[pl.ds(base + j * _CHUNK, _CHUNK)])

        rel_gather(0)
        for s in range(len(slabs) - (_NBUF - 1)):
            enqueue(s + _NBUF - 1)
            drain_wb(s)
        rel_gather(1)
        for s in range(len(slabs) - (_NBUF - 1), len(slabs)):
            drain_wb(s)

    return k(e_emb, relpad, h, t, h2, t2, r2d, r2_2d)


_TC_ROWS = 512
_TC_BLOCKS = BATCH // _TC_ROWS


def _tc_body(he, te, he2, te2, re, re2, og, ob, acc):
    p = pl.program_id(0)
    i = pl.program_id(1)
    ins = (he, te, he2, te2, re, re2)

    @pl.when((p == 0) & (i == 0))
    def _init():
        acc[...] = jnp.zeros_like(acc)

    @pl.when(p == 0)
    def _stats():
        for k, x in enumerate(ins):
            xv = x[...][:, :EMB_DIM]
            acc[2 * k, :] += jnp.sum(xv, axis=0)
            acc[2 * k + 1, :] += jnp.sum(xv * xv, axis=0)

    @pl.when(p == 1)
    def _score():
        inv_b = 1.0 / BATCH
        norm = []
        for k, x in enumerate(ins):
            m = (acc[2 * k, :] * inv_b)[None, :]
            v = (acc[2 * k + 1, :] * inv_b)[None, :] - m * m
            norm.append((x[...][:, :EMB_DIM] - m) * lax.rsqrt(v + 1e-5))
        og[...] = jnp.sum(jnp.abs(norm[0] + norm[4] - norm[1]), axis=1, keepdims=True)
        ob[...] = jnp.sum(jnp.abs(norm[2] + norm[5] - norm[3]), axis=1, keepdims=True)


def _tc_bn_score(he, te, he2, te2, re, re2):
    ent_spec = pl.BlockSpec((_TC_ROWS, EMB_DIM), lambda p, i: (i, 0))
    rel_spec = pl.BlockSpec((_TC_ROWS, 128), lambda p, i: (i, 0))
    out_spec = pl.BlockSpec((_TC_ROWS, 1), lambda p, i: (i, 0))
    return pl.pallas_call(
        _tc_body,
        grid=(2, _TC_BLOCKS),
        in_specs=[ent_spec] * 4 + [rel_spec] * 2,
        out_specs=[out_spec] * 2,
        out_shape=[jax.ShapeDtypeStruct((BATCH, 1), jnp.float32)] * 2,
        scratch_shapes=[pltpu.VMEM((12, EMB_DIM), jnp.float32)],
    )(he, te, he2, te2, re, re2)


def kernel(h, t, r, h_2, t_2, r_2, e_emb, rel_emb):
    relpad = jnp.pad(rel_emb, ((0, 0), (0, 128 - EMB_DIM)))
    i32 = lambda x: x.astype(jnp.int32)
    r2d = i32(r).reshape(_NW * _NCHUNK, _CHUNK)
    r2_2d = i32(r_2).reshape(_NW * _NCHUNK, _CHUNK)
    he, te, he2, te2, re, re2 = _sc_gather6(
        e_emb, relpad, i32(h), i32(t), i32(h_2), i32(t_2), r2d, r2_2d)
    sg, sb = _tc_bn_score(he, te, he2, te2, re, re2)
    return (sg.reshape(BATCH), sb.reshape(BATCH))
